# baseline (speedup 1.0000x reference)
import functools

import jax
import jax.numpy as jnp
from jax import lax
from jax.experimental import pallas as pl
from jax.experimental.pallas import tpu as pltpu
from jax.experimental.pallas import tpu_sc as plsc

_CHUNK = 128
_PACK = 4


@functools.lru_cache(maxsize=None)
def _make_gather(vocab: int, embed_dim: int, batch: int):
    info = plsc.get_sparse_core_info()
    num_workers = info.num_cores * info.num_subcores
    assert batch % num_workers == 0
    b_per_w = batch // num_workers
    n_chunks = b_per_w // _CHUNK
    assert b_per_w % _CHUNK == 0
    lanes = _PACK * embed_dim

    mesh = plsc.VectorSubcoreMesh(core_axis_name="c", subcore_axis_name="s")

    @functools.partial(
        pl.kernel,
        mesh=mesh,
        out_type=jax.ShapeDtypeStruct((batch, embed_dim), jnp.float32),
        compiler_params=pltpu.CompilerParams(needs_layout_passes=False),
        scratch_types=[
            pltpu.VMEM((b_per_w,), jnp.int32),
            pltpu.VMEM((n_chunks, _CHUNK), jnp.int32),
            pltpu.VMEM((2 * _CHUNK, lanes), jnp.float32),
            pltpu.VMEM((b_per_w, embed_dim), jnp.float32),
            pltpu.SemaphoreType.DMA,
            pltpu.SemaphoreType.DMA,
        ],
    )
    def gather_kernel(idx_hbm, table4_hbm, out_hbm, idx_v, idx4_v, rows4_v,
                      out_v, gsem, osem):
        wid = lax.axis_index("s") * info.num_cores + lax.axis_index("c")
        base = wid * b_per_w
        pltpu.sync_copy(idx_hbm.at[pl.ds(base, b_per_w)], idx_v)
        for j in range(n_chunks):
            for k in range(_CHUNK // 16):
                v = idx_v[pl.ds(j * _CHUNK + k * 16, 16)]
                idx4_v[j, pl.ds(k * 16, 16)] = lax.shift_right_logical(v, 2)
        def fire(j):
            return pltpu.async_copy(
                table4_hbm.at[idx4_v.at[j]],
                rows4_v.at[pl.ds((j % 2) * _CHUNK, _CHUNK)],
                gsem,
            )

        copies = [fire(0), fire(1)]
        iota = lax.iota(jnp.int32, 16)
        out_copies = []
        for j in range(n_chunks):
            copies[j].wait()
            buf = (j % 2) * _CHUNK

            def body(k, _, j=j, buf=buf):
                s = j * _CHUNK + k * 16
                idx16 = idx_v[pl.ds(s, 16)]
                row16 = iota + (buf + k * 16)
                off16 = lax.shift_left(jnp.bitwise_and(idx16, _PACK - 1), 5)
                out_row16 = iota + s
                for c in range(embed_dim):
                    vals = plsc.load_gather(rows4_v, [row16, off16 + c])
                    plsc.store_scatter(
                        out_v, [out_row16, jnp.full((16,), c, jnp.int32)], vals
                    )
                return _

            lax.fori_loop(0, _CHUNK // 16, body, 0)
            if j + 2 < n_chunks:
                copies.append(fire(j + 2))
            out_copies.append(
                pltpu.async_copy(
                    out_v.at[pl.ds(j * _CHUNK, _CHUNK)],
                    out_hbm.at[pl.ds(base + j * _CHUNK, _CHUNK)],
                    osem,
                )
            )
        for c in out_copies:
            c.wait()

    return gather_kernel, num_workers


def kernel(indices, kernel):
    table = kernel
    vocab, embed_dim = table.shape
    (batch,) = indices.shape
    gather_kernel, _ = _make_gather(vocab, embed_dim, batch)
    idx = jnp.asarray(indices, jnp.int32)
    table4 = table.reshape(vocab // _PACK, _PACK * embed_dim)
    return gather_kernel(idx, table4)


# speedup vs baseline: 1.6996x; 1.6996x over previous
import functools

import jax
import jax.numpy as jnp
from jax import lax
from jax.experimental import pallas as pl
from jax.experimental.pallas import tpu as pltpu
from jax.experimental.pallas import tpu_sc as plsc


@functools.lru_cache(maxsize=None)
def _make_gather(vocab: int, embed_dim: int, batch: int):
    info = plsc.get_sparse_core_info()
    num_workers = info.num_cores * info.num_subcores
    b_per_w = batch // num_workers

    mesh = plsc.VectorSubcoreMesh(core_axis_name="c", subcore_axis_name="s")

    @functools.partial(
        pl.kernel,
        mesh=mesh,
        out_type=jax.ShapeDtypeStruct((batch, embed_dim), jnp.float32),
        scratch_types=[
            pltpu.VMEM((b_per_w,), jnp.int32),
            pltpu.VMEM((b_per_w, embed_dim), jnp.float32),
            pltpu.SemaphoreType.DMA,
        ],
    )
    def gather_kernel(idx_hbm, table_hbm, out_hbm, idx_v, rows_v, sem):
        wid = lax.axis_index("s") * info.num_cores + lax.axis_index("c")
        base = wid * b_per_w
        pltpu.sync_copy(idx_hbm.at[pl.ds(base, b_per_w)], idx_v)

        def body(k, _):
            v = idx_v[pl.ds(k * 16, 16)]
            for l in range(16):
                r = v[l]
                pltpu.async_copy(
                    table_hbm.at[pl.ds(r, 1)],
                    rows_v.at[pl.ds(k * 16 + l, 1)],
                    sem,
                )
            return _

        lax.fori_loop(0, b_per_w // 16, body, 0)
        pltpu.make_async_copy(
            table_hbm.at[pl.ds(0, b_per_w)], rows_v, sem
        ).wait()
        pltpu.sync_copy(rows_v, out_hbm.at[pl.ds(base, b_per_w)])

    return gather_kernel


def kernel(indices, kernel):
    table = kernel
    vocab, embed_dim = table.shape
    (batch,) = indices.shape
    gather_kernel = _make_gather(vocab, embed_dim, batch)
    idx = jnp.asarray(indices, jnp.int32)
    return gather_kernel(idx, table)


# speedup vs baseline: 1.7008x; 1.0007x over previous
import functools

import jax
import jax.numpy as jnp
from jax import lax
from jax.experimental import pallas as pl
from jax.experimental.pallas import tpu as pltpu
from jax.experimental.pallas import tpu_sc as plsc


@functools.lru_cache(maxsize=None)
def _make_gather(vocab: int, embed_dim: int, batch: int):
    info = plsc.get_sparse_core_info()
    num_workers = info.num_cores * info.num_subcores
    b_per_w = batch // num_workers

    mesh = plsc.VectorSubcoreMesh(core_axis_name="c", subcore_axis_name="s")

    @functools.partial(
        pl.kernel,
        mesh=mesh,
        out_type=jax.ShapeDtypeStruct((batch, embed_dim), jnp.float32),
        compiler_params=pltpu.CompilerParams(skip_device_barrier=True),
        scratch_types=[
            pltpu.VMEM((b_per_w,), jnp.int32),
            pltpu.VMEM((b_per_w, embed_dim), jnp.float32),
            pltpu.SemaphoreType.DMA,
        ],
    )
    def gather_kernel(idx_hbm, table_hbm, out_hbm, idx_v, rows_v, sem):
        wid = lax.axis_index("s") * info.num_cores + lax.axis_index("c")
        base = wid * b_per_w
        pltpu.sync_copy(idx_hbm.at[pl.ds(base, b_per_w)], idx_v)

        def body(k, _):
            v = idx_v[pl.ds(k * 16, 16)]
            for l in range(16):
                r = v[l]
                pltpu.async_copy(
                    table_hbm.at[pl.ds(r, 1)],
                    rows_v.at[pl.ds(k * 16 + l, 1)],
                    sem,
                )
            return _

        lax.fori_loop(0, b_per_w // 16, body, 0)
        pltpu.make_async_copy(
            table_hbm.at[pl.ds(0, b_per_w)], rows_v, sem
        ).wait()
        pltpu.sync_copy(rows_v, out_hbm.at[pl.ds(base, b_per_w)])

    return gather_kernel


def kernel(indices, kernel):
    table = kernel
    vocab, embed_dim = table.shape
    (batch,) = indices.shape
    gather_kernel = _make_gather(vocab, embed_dim, batch)
    idx = jnp.asarray(indices, jnp.int32)
    return gather_kernel(idx, table)
